# baseline (device time: 54052 ns/iter reference)
import functools

import jax
import jax.numpy as jnp
from jax import lax
from jax.experimental import pallas as pl
from jax.experimental.pallas import tpu as pltpu

N_DEV = 32
N_TEAM = 16
N_CHUNK = 4
CLIP = 402.0


def kernel(x, w_mat, scale_x, scale_w):
    m_per, k = x.shape
    _, n = w_mat.shape
    n_per = n // N_DEV
    n_half = n // 2
    n_chunk = n_half // N_CHUNK
    m = m_per * N_DEV

    def body(x_ref, w_ref, sx_ref, sw_ref, out_ref,
             w_buf, xg, stage, rstage,
             load_sems, p1_send, p1_recv, p2_send, p2_recv):
        p = lax.axis_index("i")
        z = p // 8
        hp = z % 2
        h = p // N_TEAM
        tm = h * 8 + p % 8
        jz = z % 2
        partner = p ^ 8

        def start_load(j):
            cp = pltpu.make_async_copy(
                w_ref.at[:, pl.ds(hp * n_half + j * n_chunk, n_chunk)],
                w_buf.at[j % 2],
                load_sems.at[j % 2])
            cp.start()
            return cp

        loads = [start_load(0), start_load(1)]

        xg[jz, :, :] = x_ref[:, :].astype(jnp.float8_e5m2)

        barrier = pltpu.get_barrier_semaphore()
        for dd in range(1, N_DEV):
            pl.semaphore_signal(
                barrier, inc=1, device_id=((p + dd) % N_DEV,),
                device_id_type=pl.DeviceIdType.MESH)
        pl.semaphore_wait(barrier, N_DEV - 1)

        p1 = pltpu.make_async_remote_copy(
            src_ref=xg.at[jz], dst_ref=xg.at[jz],
            send_sem=p1_send, recv_sem=p1_recv.at[jz],
            device_id=(partner,), device_id_type=pl.DeviceIdType.MESH)
        p1.start()

        inv_step = jnp.float32(127.0 / CLIP)
        p2_rdmas = []

        p1r = pltpu.make_async_remote_copy(
            src_ref=xg.at[jz], dst_ref=xg.at[1 - jz],
            send_sem=p1_send, recv_sem=p1_recv.at[1 - jz],
            device_id=(partner,), device_id_type=pl.DeviceIdType.MESH)
        p1r.wait_recv()
        xall = xg[:, :, :].reshape(2 * m_per, k).astype(jnp.float32)

        for j in range(N_CHUNK):
            loads[j].wait()
            acc = jnp.dot(xall, w_buf[j % 2],
                          preferred_element_type=jnp.float32)
            q = jnp.clip(jnp.round(acc * inv_step), -127, 127)
            stage[:, pl.ds(j * n_chunk, n_chunk)] = q.astype(jnp.int8)
            for rr in range(4):
                d16 = j * 4 + rr
                dest = 16 * hp + d16
                for js in range(2):
                    @pl.when(dest != p)
                    def _(js=js, d16=d16, dest=dest):
                        rdma = pltpu.make_async_remote_copy(
                            src_ref=stage.at[pl.ds(js * m_per, m_per),
                                             pl.ds(d16 * n_per, n_per)],
                            dst_ref=rstage.at[js, tm],
                            send_sem=p2_send.at[js, d16],
                            recv_sem=p2_recv.at[js, tm],
                            device_id=(dest,),
                            device_id_type=pl.DeviceIdType.MESH)
                        rdma.start()
                    p2_rdmas.append((js, d16, dest))
            if j + 2 < N_CHUNK:
                loads.append(start_load(j + 2))

        dq = jnp.float32(CLIP / 127.0) * sx_ref[0] * sw_ref[0]

        own_d16 = p - 16 * hp

        @pl.when((own_d16 >= 0) & (own_d16 < 16))
        def _():
            for js in range(2):
                row_dev = (tm // 8) * 16 + tm % 8 + 8 * js
                out_ref[pl.ds(row_dev * m_per, m_per), :] = (
                    stage[pl.ds(js * m_per, m_per),
                          pl.ds(own_d16 * n_per, n_per)]
                    .astype(jnp.float32) * dq)

        for t in range(N_TEAM):
            p0 = 16 * (t // 8) + t % 8
            for js in range(2):
                sender = (2 * (t // 8) + h) * 8 + t % 8

                @pl.when(sender != p)
                def _(t=t, js=js, p0=p0):
                    recv = pltpu.make_async_remote_copy(
                        src_ref=stage.at[pl.ds(js * m_per, m_per),
                                         pl.ds(0, n_per)],
                        dst_ref=rstage.at[js, t],
                        send_sem=p2_send.at[js, 0],
                        recv_sem=p2_recv.at[js, t],
                        device_id=(0,),
                        device_id_type=pl.DeviceIdType.MESH)
                    recv.wait_recv()
                    out_ref[pl.ds((p0 + 8 * js) * m_per, m_per), :] = (
                        rstage[js, t].astype(jnp.float32) * dq)

        p1.wait_send()
        for js, d16, dest in p2_rdmas:
            @pl.when(dest != p)
            def _(js=js, d16=d16, dest=dest):
                w = pltpu.make_async_remote_copy(
                    src_ref=stage.at[pl.ds(js * m_per, m_per),
                                     pl.ds(d16 * n_per, n_per)],
                    dst_ref=rstage.at[js, tm],
                    send_sem=p2_send.at[js, d16],
                    recv_sem=p2_recv.at[js, tm],
                    device_id=(dest,),
                    device_id_type=pl.DeviceIdType.MESH)
                w.wait_send()

        @functools.partial(
            pl.run_scoped, exit_sem=pltpu.SemaphoreType.REGULAR)
        def _(exit_sem):
            for dd in range(1, N_DEV):
                pl.semaphore_signal(
                    exit_sem, inc=1, device_id=((p + dd) % N_DEV,),
                    device_id_type=pl.DeviceIdType.MESH)
            pl.semaphore_wait(exit_sem, N_DEV - 1)

    return pl.pallas_call(
        body,
        out_shape=jax.ShapeDtypeStruct((m, n_per), jnp.float32),
        in_specs=[
            pl.BlockSpec(memory_space=pltpu.VMEM),
            pl.BlockSpec(memory_space=pl.ANY),
            pl.BlockSpec(memory_space=pltpu.SMEM),
            pl.BlockSpec(memory_space=pltpu.SMEM),
        ],
        out_specs=pl.BlockSpec(memory_space=pltpu.VMEM),
        scratch_shapes=[
            pltpu.VMEM((2, k, n_chunk), jnp.float32),
            pltpu.VMEM((2, m_per, k), jnp.float8_e5m2),
            pltpu.VMEM((2 * m_per, n_half), jnp.int8),
            pltpu.VMEM((2, N_TEAM, m_per, n_per), jnp.int8),
            pltpu.SemaphoreType.DMA((2,)),
            pltpu.SemaphoreType.DMA,
            pltpu.SemaphoreType.DMA((2,)),
            pltpu.SemaphoreType.DMA((2, N_TEAM)),
            pltpu.SemaphoreType.DMA((2, N_TEAM)),
        ],
        compiler_params=pltpu.CompilerParams(
            collective_id=0,
            vmem_limit_bytes=60 * 1024 * 1024,
        ),
    )(x, w_mat, scale_x, scale_w)


# device time: 49624 ns/iter; 1.0892x vs baseline; 1.0892x over previous
import functools
import os

_KMODE = os.environ.get("KMODE", "full")

import jax
import jax.numpy as jnp
from jax import lax
from jax.experimental import pallas as pl
from jax.experimental.pallas import tpu as pltpu

N_DEV = 32
N_TEAM = 16
N_CHUNK = 4
CLIP = 402.0


def kernel(x, w_mat, scale_x, scale_w):
    m_per, k = x.shape
    _, n = w_mat.shape
    n_per = n // N_DEV
    n_half = n // 2
    n_chunk = n_half // N_CHUNK
    m = m_per * N_DEV

    def body(x_ref, w_ref, sx_ref, sw_ref, out_ref,
             w_buf, xg, stage, rstage,
             load_sems, p1_send, p1_recv, p2_send, p2_recv):
        p = lax.axis_index("i")
        z = p // 8
        hp = z % 2
        h = p // N_TEAM
        tm = h * 8 + p % 8
        jz = z % 2
        partner = p ^ 8

        rot = p % N_CHUNK

        def start_load(j):
            j_eff = (j + p) % N_CHUNK
            cp = pltpu.make_async_copy(
                w_ref.at[:, pl.ds(hp * n_half + j_eff * n_chunk, n_chunk)],
                w_buf.at[j % 2],
                load_sems.at[j % 2])
            cp.start()
            return cp

        loads = [start_load(0), start_load(1)]

        if _KMODE == "loadonly":
            for j in range(N_CHUNK):
                loads[j].wait()
                if j + 2 < N_CHUNK:
                    loads.append(start_load(j + 2))
            out_ref[0:1, 0:1] = jnp.zeros((1, 1), jnp.float32)
            return

        xg[jz, :, :] = x_ref[:, :].astype(jnp.float8_e5m2)

        if _KMODE == "nop1":
            xg[1 - jz, :, :] = x_ref[:, :].astype(jnp.float8_e5m2)
        barrier = pltpu.get_barrier_semaphore()
        for dd in (range(1, N_DEV) if _KMODE != "nop1" else []):
            pl.semaphore_signal(
                barrier, inc=1, device_id=((p + dd) % N_DEV,),
                device_id_type=pl.DeviceIdType.MESH)
        if _KMODE != "nop1":
            pl.semaphore_wait(barrier, N_DEV - 1)

        if _KMODE != "nop1":
            p1 = pltpu.make_async_remote_copy(
                src_ref=xg.at[jz], dst_ref=xg.at[jz],
                send_sem=p1_send, recv_sem=p1_recv.at[jz],
                device_id=(partner,), device_id_type=pl.DeviceIdType.MESH)
            p1.start()

        inv_step = jnp.float32(127.0 / CLIP)
        p2_rdmas = []

        x_own = x_ref[:, :].astype(jnp.bfloat16)
        xp = [None]

        def do_shard(js, x_bf, j_eff, wb):
            acc = jnp.dot(x_bf, wb, preferred_element_type=jnp.float32)
            q = jnp.clip(jnp.round(acc * inv_step), -127, 127)
            stage[pl.ds(js * m_per, m_per),
                  pl.ds(j_eff * n_chunk, n_chunk)] = q.astype(jnp.int8)
            if _KMODE != "full":
                return
            for rr in range(4):
                d16 = j_eff * 4 + (rr + p) % 4
                dest = 16 * hp + d16

                @pl.when(dest != p)
                def _(js=js, d16=d16, dest=dest):
                    rdma = pltpu.make_async_remote_copy(
                        src_ref=stage.at[pl.ds(js * m_per, m_per),
                                         pl.ds(d16 * n_per, n_per)],
                        dst_ref=rstage.at[js, tm],
                        send_sem=p2_send.at[js, d16],
                        recv_sem=p2_recv.at[js, tm],
                        device_id=(dest,),
                        device_id_type=pl.DeviceIdType.MESH)
                    rdma.start()
                p2_rdmas.append((js, d16, dest))

        for jj in range(N_CHUNK):
            j_eff = (jj + p) % N_CHUNK
            loads[jj].wait()
            wb = w_buf[jj % 2].astype(jnp.bfloat16)
            do_shard(jz, x_own, j_eff, wb)
            if jj == 0:
                if _KMODE != "nop1":
                    p1r = pltpu.make_async_remote_copy(
                        src_ref=xg.at[jz], dst_ref=xg.at[1 - jz],
                        send_sem=p1_send, recv_sem=p1_recv.at[1 - jz],
                        device_id=(partner,),
                        device_id_type=pl.DeviceIdType.MESH)
                    p1r.wait_recv()
                xp[0] = xg[1 - jz, :, :].astype(jnp.bfloat16)
            do_shard(1 - jz, xp[0], j_eff, wb)
            if jj + 2 < N_CHUNK:
                loads.append(start_load(jj + 2))

        dq = jnp.float32(CLIP / 127.0) * sx_ref[0] * sw_ref[0]

        own_d16 = p - 16 * hp

        @pl.when((own_d16 >= 0) & (own_d16 < 16))
        def _():
            for js in range(2):
                row_dev = (tm // 8) * 16 + tm % 8 + 8 * js
                out_ref[pl.ds(row_dev * m_per, m_per), :] = (
                    stage[pl.ds(js * m_per, m_per),
                          pl.ds(own_d16 * n_per, n_per)]
                    .astype(jnp.float32) * dq)

        for t in (range(N_TEAM) if _KMODE == "full" else []):
            p0 = 16 * (t // 8) + t % 8
            for js in range(2):
                sender = (2 * (t // 8) + h) * 8 + t % 8

                @pl.when(sender != p)
                def _(t=t, js=js, p0=p0):
                    recv = pltpu.make_async_remote_copy(
                        src_ref=stage.at[pl.ds(js * m_per, m_per),
                                         pl.ds(0, n_per)],
                        dst_ref=rstage.at[js, t],
                        send_sem=p2_send.at[js, 0],
                        recv_sem=p2_recv.at[js, t],
                        device_id=(0,),
                        device_id_type=pl.DeviceIdType.MESH)
                    recv.wait_recv()
                    out_ref[pl.ds((p0 + 8 * js) * m_per, m_per), :] = (
                        rstage[js, t].astype(jnp.float32) * dq)

        if _KMODE != "nop1":
            p1.wait_send()
        for js, d16, dest in p2_rdmas:
            @pl.when(dest != p)
            def _(js=js, d16=d16, dest=dest):
                w = pltpu.make_async_remote_copy(
                    src_ref=stage.at[pl.ds(js * m_per, m_per),
                                     pl.ds(d16 * n_per, n_per)],
                    dst_ref=rstage.at[js, tm],
                    send_sem=p2_send.at[js, d16],
                    recv_sem=p2_recv.at[js, tm],
                    device_id=(dest,),
                    device_id_type=pl.DeviceIdType.MESH)
                w.wait_send()

        if _KMODE != "full":
            return
        @functools.partial(
            pl.run_scoped, exit_sem=pltpu.SemaphoreType.REGULAR)
        def _(exit_sem):
            for dd in range(1, N_DEV):
                pl.semaphore_signal(
                    exit_sem, inc=1, device_id=((p + dd) % N_DEV,),
                    device_id_type=pl.DeviceIdType.MESH)
            pl.semaphore_wait(exit_sem, N_DEV - 1)

    return pl.pallas_call(
        body,
        out_shape=jax.ShapeDtypeStruct((m, n_per), jnp.float32),
        in_specs=[
            pl.BlockSpec(memory_space=pltpu.VMEM),
            pl.BlockSpec(memory_space=pl.ANY),
            pl.BlockSpec(memory_space=pltpu.SMEM),
            pl.BlockSpec(memory_space=pltpu.SMEM),
        ],
        out_specs=pl.BlockSpec(memory_space=pltpu.VMEM),
        scratch_shapes=[
            pltpu.VMEM((2, k, n_chunk), jnp.float32),
            pltpu.VMEM((2, m_per, k), jnp.float8_e5m2),
            pltpu.VMEM((2 * m_per, n_half), jnp.int8),
            pltpu.VMEM((2, N_TEAM, m_per, n_per), jnp.int8),
            pltpu.SemaphoreType.DMA((2,)),
            pltpu.SemaphoreType.DMA,
            pltpu.SemaphoreType.DMA((2,)),
            pltpu.SemaphoreType.DMA((2, N_TEAM)),
            pltpu.SemaphoreType.DMA((2, N_TEAM)),
        ],
        compiler_params=pltpu.CompilerParams(
            collective_id=None if _KMODE in ("loadonly", "nop1") else 0,
            vmem_limit_bytes=60 * 1024 * 1024,
        ),
    )(x, w_mat, scale_x, scale_w)
